# baseline (speedup 1.0000x reference)
import jax
import jax.numpy as jnp
from jax import lax
from jax.experimental import pallas as pl
from jax.experimental.pallas import tpu as pltpu
from jax.experimental.pallas import tpu_sc as plsc

N = 10000
E = 320000
D = 128
HID = 128
LV = 100

NP = 10240
NC = 2
NS = 16
NW = NC * NS
EC = 128
EPW_C = 80
EPW = EPW_C * EC
EP = NW * EPW
EPW16_C = 2 * EPW_C
RPT = NP // NS
F32 = jnp.float32
I32 = jnp.int32

_mesh = lambda: plsc.VectorSubcoreMesh(
    core_axis_name="c", subcore_axis_name="s", num_cores=NC, num_subcores=NS)


def _deg_body(dst_hbm, ew_hbm, dinv_hbm, dstb, ewb, vecb, deg_sh):
    c = lax.axis_index("c")
    t = lax.axis_index("s")

    @pl.when(c == 0)
    def _():
        def fill(j, carry):
            vecb[pl.ds(j * 16, 16)] = jnp.full((16,), 1.0, F32)
            return carry
        lax.fori_loop(0, RPT // 16, fill, None)
        pltpu.sync_copy(vecb, deg_sh.at[pl.ds(t * RPT, RPT)])

    plsc.subcore_barrier()

    @pl.when(c == 0)
    def _():
        pltpu.sync_copy(dst_hbm.at[t], dstb)
        pltpu.sync_copy(ew_hbm.at[t], ewb)

        def body(j, carry):
            pltpu.sync_copy(ewb.at[j], deg_sh.at[dstb.at[j]], add=True)
            return carry
        lax.fori_loop(0, EPW16_C, body, None)

    plsc.subcore_barrier()

    @pl.when(c == 0)
    def _():
        pltpu.sync_copy(deg_sh.at[pl.ds(t * RPT, RPT)],
                        dinv_hbm.at[pl.ds(t * RPT, RPT)])


def _sc_deg(dst16, ew16):
    return pl.kernel(
        _deg_body,
        out_type=jax.ShapeDtypeStruct((NP,), F32),
        mesh=_mesh(),
        compiler_params=pltpu.CompilerParams(needs_layout_passes=False),
        scratch_types=[
            pltpu.VMEM((EPW16_C, EC), I32),
            pltpu.VMEM((EPW16_C, EC), F32),
            pltpu.VMEM((RPT,), F32),
            pltpu.VMEM_SHARED((NP,), F32),
        ],
    )(dst16, ew16)


def _agg_body(src_hbm, dst_hbm, ew_hbm, dinv_hbm, h_hbm, out_hbm,
              esrc, edst, eew, normc, rows, dinvv,
              seme, semg, acc):
    c = lax.axis_index("c")
    t = lax.axis_index("s")
    wid = t * NC + c

    pltpu.sync_copy(dinv_hbm, dinvv)

    def zrow(r, carry):
        for cc in range(8):
            rows[r, pl.ds(cc * 16, 16)] = jnp.zeros((16,), F32)
        return carry
    lax.fori_loop(0, EC, zrow, None)
    for k in range(RPT // EC):
        pltpu.sync_copy(rows, acc.at[pl.ds(t * RPT + k * EC, EC)])

    plsc.subcore_barrier()

    def chunk(j, carry):
        d1 = pltpu.async_copy(src_hbm.at[wid, j], esrc, seme)
        d2 = pltpu.async_copy(dst_hbm.at[wid, j], edst, seme)
        d3 = pltpu.async_copy(ew_hbm.at[wid, j], eew, seme)
        d1.wait(); d2.wait(); d3.wait()
        dg = pltpu.async_copy(h_hbm.at[esrc], rows, semg)
        for v in range(8):
            sl = pl.ds(v * 16, 16)
            normc[sl] = (eew[sl]
                         * plsc.load_gather(dinvv, [esrc[sl]])
                         * plsc.load_gather(dinvv, [edst[sl]]))
        dg.wait()
        for e in range(EC):
            sp = plsc.load_gather(normc, [jnp.full((16,), e, I32)])
            for cc in range(8):
                csl = pl.ds(cc * 16, 16)
                rows[e, csl] = rows[e, csl] * sp
        pltpu.sync_copy(rows, acc.at[edst], add=True)
        return carry
    lax.fori_loop(0, EPW_C, chunk, None)

    plsc.subcore_barrier()

    for k in range(RPT // EC):
        sl = pl.ds(t * RPT + k * EC, EC)
        pltpu.sync_copy(acc.at[sl], out_hbm.at[c, sl])


def _sc_aggregate(src32, dst32, ew32, dinv, h):
    return pl.kernel(
        _agg_body,
        out_type=jax.ShapeDtypeStruct((NC, NP, HID), F32),
        mesh=_mesh(),
        compiler_params=pltpu.CompilerParams(needs_layout_passes=False),
        scratch_types=[
            pltpu.VMEM((EC,), I32),
            pltpu.VMEM((EC,), I32),
            pltpu.VMEM((EC,), F32),
            pltpu.VMEM((EC,), F32),
            pltpu.VMEM((EC, HID), F32),
            pltpu.VMEM((NP,), F32),
            pltpu.SemaphoreType.DMA, pltpu.SemaphoreType.DMA,
            pltpu.VMEM_SHARED((NP, HID), F32),
        ],
    )(src32, dst32, ew32, dinv, h)


def _rsqrt_body(d_ref, o_ref):
    o_ref[...] = lax.rsqrt(d_ref[...])


def _tc_rsqrt(deg2d):
    return pl.pallas_call(
        _rsqrt_body,
        out_shape=jax.ShapeDtypeStruct((NP // 128, 128), F32),
    )(deg2d)


def _mm_body(x_ref, w_ref, o_ref):
    o_ref[...] = jnp.dot(x_ref[...], w_ref[...], preferred_element_type=F32)


def _tc_matmul(x_p, W1):
    return pl.pallas_call(
        _mm_body,
        grid=(NP // 512,),
        in_specs=[
            pl.BlockSpec((512, D), lambda i: (i, 0)),
            pl.BlockSpec((D, HID), lambda i: (0, 0)),
        ],
        out_specs=pl.BlockSpec((512, HID), lambda i: (i, 0)),
        out_shape=jax.ShapeDtypeStruct((NP, HID), F32),
    )(x_p, W1)


def _zall_body(p0_ref, p1_ref, dv_ref, h_ref, b_ref, o_ref):
    dv = dv_ref[...]
    o_ref[...] = p0_ref[...] + p1_ref[...] + (dv * dv) * h_ref[...] + b_ref[...]


def _tc_zall(p0, p1, dinv_col, h, b1_row):
    blk = 512
    return pl.pallas_call(
        _zall_body,
        grid=(NP // blk,),
        in_specs=[
            pl.BlockSpec((blk, HID), lambda i: (i, 0)),
            pl.BlockSpec((blk, HID), lambda i: (i, 0)),
            pl.BlockSpec((blk, 1), lambda i: (i, 0)),
            pl.BlockSpec((blk, HID), lambda i: (i, 0)),
            pl.BlockSpec((1, HID), lambda i: (0, 0)),
        ],
        out_specs=pl.BlockSpec((blk, HID), lambda i: (i, 0)),
        out_shape=jax.ShapeDtypeStruct((NP, HID), F32),
    )(p0, p1, dinv_col, h, b1_row)


def _lstm_body(vis_ref, z_ref, wi0, wh0, bi0, bh0, wi1, wh1, bi1, bh1,
               out_ref, h0, c0, h1, c1):
    i = pl.program_id(0)

    @pl.when(i == 0)
    def _():
        h0[...] = jnp.zeros((1, HID), F32)
        c0[...] = jnp.zeros((1, HID), F32)
        h1[...] = jnp.zeros((1, HID), F32)
        c1[...] = jnp.zeros((1, HID), F32)

    x = z_ref[0]
    g = (jnp.dot(x, wi0[...], preferred_element_type=F32)
         + jnp.dot(h0[...], wh0[...], preferred_element_type=F32)
         + bi0[...] + bh0[...])
    ii = jax.nn.sigmoid(g[:, 0:HID])
    ff = jax.nn.sigmoid(g[:, HID:2 * HID])
    gg = jnp.tanh(g[:, 2 * HID:3 * HID])
    oo = jax.nn.sigmoid(g[:, 3 * HID:4 * HID])
    cn0 = ff * c0[...] + ii * gg
    c0[...] = cn0
    hn0 = oo * jnp.tanh(cn0)
    h0[...] = hn0

    g1 = (jnp.dot(hn0, wi1[...], preferred_element_type=F32)
          + jnp.dot(h1[...], wh1[...], preferred_element_type=F32)
          + bi1[...] + bh1[...])
    ii1 = jax.nn.sigmoid(g1[:, 0:HID])
    ff1 = jax.nn.sigmoid(g1[:, HID:2 * HID])
    gg1 = jnp.tanh(g1[:, 2 * HID:3 * HID])
    oo1 = jax.nn.sigmoid(g1[:, 3 * HID:4 * HID])
    cn1 = ff1 * c1[...] + ii1 * gg1
    c1[...] = cn1
    hn1 = oo1 * jnp.tanh(cn1)
    h1[...] = hn1

    @pl.when(i == LV - 1)
    def _():
        out_ref[...] = hn1


def _tc_lstm(visited, z_all, wi0, wh0, bi0, bh0, wi1, wh1, bi1, bh1):
    wspec = pl.BlockSpec((HID, 4 * HID), lambda i, vis: (0, 0))
    bspec = pl.BlockSpec((1, 4 * HID), lambda i, vis: (0, 0))
    grid_spec = pltpu.PrefetchScalarGridSpec(
        num_scalar_prefetch=1,
        grid=(LV,),
        in_specs=[
            pl.BlockSpec((1, 1, HID), lambda i, vis: (vis[i], 0, 0)),
            wspec, wspec, bspec, bspec, wspec, wspec, bspec, bspec,
        ],
        out_specs=pl.BlockSpec((1, HID), lambda i, vis: (0, 0)),
        scratch_shapes=[pltpu.VMEM((1, HID), F32)] * 4,
    )
    return pl.pallas_call(
        _lstm_body,
        grid_spec=grid_spec,
        out_shape=jax.ShapeDtypeStruct((1, HID), F32),
    )(visited, z_all, wi0, wh0, bi0, bh0, wi1, wh1, bi1, bh1)


def _score_body(vis_ref, z3_ref, fh_ref, g_ref, probs_ref, idx_ref):
    z = z3_ref[...]
    fh = fh_ref[...]
    s = jnp.sum(z * fh[0][None, None, :], axis=2)
    row = lax.broadcasted_iota(I32, (NP // 128, 128), 0)
    col = lax.broadcasted_iota(I32, (NP // 128, 128), 1)
    gidx = row * 128 + col

    s = jnp.where(gidx >= N, -jnp.inf, s)

    def mk(j, sv):
        return jnp.where(gidx == vis_ref[j], -jnp.inf, sv)
    s = lax.fori_loop(0, LV, mk, s)

    mx = jnp.max(s)
    ex = jnp.exp(s - mx)
    probs_ref[...] = ex / jnp.sum(ex)

    v2 = s + g_ref[...]
    mx2 = jnp.max(v2)
    flat = jnp.where(v2 == mx2, gidx, jnp.int32(2 ** 30))
    idx_ref[...] = jnp.full((1, 1), jnp.min(flat), I32)


def _tc_scores(visited, z3, fh, g2d):
    grid_spec = pltpu.PrefetchScalarGridSpec(
        num_scalar_prefetch=1,
        grid=(1,),
        in_specs=[
            pl.BlockSpec((NP // 128, 128, 128), lambda i, vis: (0, 0, 0)),
            pl.BlockSpec((1, HID), lambda i, vis: (0, 0)),
            pl.BlockSpec((NP // 128, 128), lambda i, vis: (0, 0)),
        ],
        out_specs=[
            pl.BlockSpec((NP // 128, 128), lambda i, vis: (0, 0)),
            pl.BlockSpec((1, 1), lambda i, vis: (0, 0)),
        ],
    )
    return pl.pallas_call(
        _score_body,
        grid_spec=grid_spec,
        out_shape=[
            jax.ShapeDtypeStruct((NP // 128, 128), F32),
            jax.ShapeDtypeStruct((1, 1), I32),
        ],
    )(visited, z3, fh, g2d)


def kernel(x, edge_index, edge_weight, visited,
           W1, b1, W_ih0, W_hh0, b_ih0, b_hh0, W_ih1, W_hh1, b_ih1, b_hh1):
    src = edge_index[0]
    dst = edge_index[1]
    pad = EP - E
    srcp = jnp.concatenate([src, jnp.zeros((pad,), I32)])
    dstp = jnp.concatenate([dst, jnp.zeros((pad,), I32)])
    ewp = jnp.concatenate([edge_weight, jnp.zeros((pad,), F32)])

    src32 = srcp.reshape(NW, EPW_C, EC)
    dst32 = dstp.reshape(NW, EPW_C, EC)
    ew32 = ewp.reshape(NW, EPW_C, EC)
    dst16 = dstp.reshape(NS, EPW16_C, EC)
    ew16 = ewp.reshape(NS, EPW16_C, EC)

    x_p = jnp.concatenate([x, jnp.zeros((NP - N, D), F32)])

    deg = _sc_deg(dst16, ew16)
    dinv = _tc_rsqrt(deg.reshape(NP // 128, 128)).reshape(NP)
    h = _tc_matmul(x_p, W1)
    parts = _sc_aggregate(src32, dst32, ew32, dinv, h)
    z_all = _tc_zall(parts[0], parts[1], dinv.reshape(NP, 1), h,
                     b1.reshape(1, HID))

    fh = _tc_lstm(visited, z_all.reshape(NP, 1, HID),
                  W_ih0.T, W_hh0.T, b_ih0.reshape(1, 4 * HID),
                  b_hh0.reshape(1, 4 * HID),
                  W_ih1.T, W_hh1.T, b_ih1.reshape(1, 4 * HID),
                  b_hh1.reshape(1, 4 * HID))

    g = jax.random.gumbel(jax.random.key(1), (N,), F32)
    g2d = jnp.concatenate([g, jnp.zeros((NP - N,), F32)]).reshape(NP // 128, 128)

    probs2d, idx = _tc_scores(visited, z_all.reshape(NP // 128, 128, 128),
                              fh, g2d)
    probs = probs2d.reshape(NP)[:N]
    next_node = idx[0, 0]
    return (next_node, probs)


# speedup vs baseline: 1.6428x; 1.6428x over previous
import jax
import jax.numpy as jnp
from jax import lax
from jax.experimental import pallas as pl
from jax.experimental.pallas import tpu as pltpu
from jax.experimental.pallas import tpu_sc as plsc

N = 10000
E = 320000
D = 128
HID = 128
LV = 100

NP = 10240
NC = 2
NS = 16
NW = NC * NS
EC = 128
EPW_C = 80
EPW = EPW_C * EC
EP = NW * EPW
EPW16_C = 2 * EPW_C
RPT = NP // NS
F32 = jnp.float32
I32 = jnp.int32

_mesh = lambda: plsc.VectorSubcoreMesh(
    core_axis_name="c", subcore_axis_name="s", num_cores=NC, num_subcores=NS)


def _deg_body(dst_hbm, ew_hbm, dinv_hbm, dstb, ewb, vecb, deg_sh):
    c = lax.axis_index("c")
    t = lax.axis_index("s")

    @pl.when(c == 0)
    def _():
        def fill(j, carry):
            vecb[pl.ds(j * 16, 16)] = jnp.full((16,), 1.0, F32)
            return carry
        lax.fori_loop(0, RPT // 16, fill, None)
        pltpu.sync_copy(vecb, deg_sh.at[pl.ds(t * RPT, RPT)])

    plsc.subcore_barrier()

    @pl.when(c == 0)
    def _():
        pltpu.sync_copy(dst_hbm.at[t], dstb)
        pltpu.sync_copy(ew_hbm.at[t], ewb)

        def body(j, carry):
            pltpu.sync_copy(ewb.at[j], deg_sh.at[dstb.at[j]], add=True)
            return carry
        lax.fori_loop(0, EPW16_C, body, None)

    plsc.subcore_barrier()

    @pl.when(c == 0)
    def _():
        pltpu.sync_copy(deg_sh.at[pl.ds(t * RPT, RPT)],
                        dinv_hbm.at[pl.ds(t * RPT, RPT)])


def _sc_deg(dst16, ew16):
    return pl.kernel(
        _deg_body,
        out_type=jax.ShapeDtypeStruct((NP,), F32),
        mesh=_mesh(),
        compiler_params=pltpu.CompilerParams(needs_layout_passes=False),
        scratch_types=[
            pltpu.VMEM((EPW16_C, EC), I32),
            pltpu.VMEM((EPW16_C, EC), F32),
            pltpu.VMEM((RPT,), F32),
            pltpu.VMEM_SHARED((NP,), F32),
        ],
    )(dst16, ew16)


def _agg_body(src_hbm, dst_hbm, ew_hbm, dinv_hbm, h_hbm, out_hbm,
              esrc, edst, eew, normc, rows, dinvv,
              seme, semg, acc):
    c = lax.axis_index("c")
    t = lax.axis_index("s")
    wid = t * NC + c

    pltpu.sync_copy(dinv_hbm, dinvv)

    def zrow(r, carry):
        for cc in range(8):
            rows[r, pl.ds(cc * 16, 16)] = jnp.zeros((16,), F32)
        return carry
    lax.fori_loop(0, EC, zrow, None)
    for k in range(RPT // EC):
        pltpu.sync_copy(rows, acc.at[pl.ds(t * RPT + k * EC, EC)])

    plsc.subcore_barrier()

    def chunk(j, carry):
        d1 = pltpu.async_copy(src_hbm.at[wid, j], esrc, seme)
        d2 = pltpu.async_copy(dst_hbm.at[wid, j], edst, seme)
        d3 = pltpu.async_copy(ew_hbm.at[wid, j], eew, seme)
        d1.wait(); d2.wait(); d3.wait()
        dg = pltpu.async_copy(h_hbm.at[esrc], rows, semg)
        for v in range(8):
            sl = pl.ds(v * 16, 16)
            normc[sl] = (eew[sl]
                         * plsc.load_gather(dinvv, [esrc[sl]])
                         * plsc.load_gather(dinvv, [edst[sl]]))
        dg.wait()
        for e in range(EC):
            sp = plsc.load_gather(normc, [jnp.full((16,), e, I32)])
            for cc in range(8):
                csl = pl.ds(cc * 16, 16)
                rows[e, csl] = rows[e, csl] * sp
        pltpu.sync_copy(rows, acc.at[edst], add=True)
        return carry
    lax.fori_loop(0, EPW_C, chunk, None)

    plsc.subcore_barrier()

    for k in range(RPT // EC):
        sl = pl.ds(t * RPT + k * EC, EC)
        pltpu.sync_copy(acc.at[sl], out_hbm.at[c, sl])


def _sc_aggregate(src32, dst32, ew32, dinv, h):
    return pl.kernel(
        _agg_body,
        out_type=jax.ShapeDtypeStruct((NC, NP, HID), F32),
        mesh=_mesh(),
        compiler_params=pltpu.CompilerParams(needs_layout_passes=False),
        scratch_types=[
            pltpu.VMEM((EC,), I32),
            pltpu.VMEM((EC,), I32),
            pltpu.VMEM((EC,), F32),
            pltpu.VMEM((EC,), F32),
            pltpu.VMEM((EC, HID), F32),
            pltpu.VMEM((NP,), F32),
            pltpu.SemaphoreType.DMA, pltpu.SemaphoreType.DMA,
            pltpu.VMEM_SHARED((NP, HID), F32),
        ],
    )(src32, dst32, ew32, dinv, h)


def _rsqrt_body(d_ref, o_ref):
    o_ref[...] = lax.rsqrt(d_ref[...])


def _tc_rsqrt(deg2d):
    return pl.pallas_call(
        _rsqrt_body,
        out_shape=jax.ShapeDtypeStruct((NP // 128, 128), F32),
    )(deg2d)


def _mm_body(x_ref, w_ref, o_ref):
    o_ref[...] = jnp.dot(x_ref[...], w_ref[...], preferred_element_type=F32)


def _tc_matmul(x_p, W1):
    return pl.pallas_call(
        _mm_body,
        grid=(NP // 512,),
        in_specs=[
            pl.BlockSpec((512, D), lambda i: (i, 0)),
            pl.BlockSpec((D, HID), lambda i: (0, 0)),
        ],
        out_specs=pl.BlockSpec((512, HID), lambda i: (i, 0)),
        out_shape=jax.ShapeDtypeStruct((NP, HID), F32),
    )(x_p, W1)


def _zall_body(p0_ref, p1_ref, dv_ref, h_ref, b_ref, o_ref):
    dv = dv_ref[...]
    o_ref[...] = p0_ref[...] + p1_ref[...] + (dv * dv) * h_ref[...] + b_ref[...]


def _tc_zall(p0, p1, dinv_col, h, b1_row):
    blk = 512
    return pl.pallas_call(
        _zall_body,
        grid=(NP // blk,),
        in_specs=[
            pl.BlockSpec((blk, HID), lambda i: (i, 0)),
            pl.BlockSpec((blk, HID), lambda i: (i, 0)),
            pl.BlockSpec((blk, 1), lambda i: (i, 0)),
            pl.BlockSpec((blk, HID), lambda i: (i, 0)),
            pl.BlockSpec((1, HID), lambda i: (0, 0)),
        ],
        out_specs=pl.BlockSpec((blk, HID), lambda i: (i, 0)),
        out_shape=jax.ShapeDtypeStruct((NP, HID), F32),
    )(p0, p1, dinv_col, h, b1_row)


def _lstm_body(vis_ref, z_ref, wi0, wh0, bi0, bh0, wi1, wh1, bi1, bh1,
               out_ref, h0, c0, h1, c1):
    i = pl.program_id(0)

    @pl.when(i == 0)
    def _():
        h0[...] = jnp.zeros((1, HID), F32)
        c0[...] = jnp.zeros((1, HID), F32)
        h1[...] = jnp.zeros((1, HID), F32)
        c1[...] = jnp.zeros((1, HID), F32)

    x = z_ref[0]
    g = (jnp.dot(x, wi0[...], preferred_element_type=F32)
         + jnp.dot(h0[...], wh0[...], preferred_element_type=F32)
         + bi0[...] + bh0[...])
    ii = jax.nn.sigmoid(g[:, 0:HID])
    ff = jax.nn.sigmoid(g[:, HID:2 * HID])
    gg = jnp.tanh(g[:, 2 * HID:3 * HID])
    oo = jax.nn.sigmoid(g[:, 3 * HID:4 * HID])
    cn0 = ff * c0[...] + ii * gg
    c0[...] = cn0
    hn0 = oo * jnp.tanh(cn0)
    h0[...] = hn0

    g1 = (jnp.dot(hn0, wi1[...], preferred_element_type=F32)
          + jnp.dot(h1[...], wh1[...], preferred_element_type=F32)
          + bi1[...] + bh1[...])
    ii1 = jax.nn.sigmoid(g1[:, 0:HID])
    ff1 = jax.nn.sigmoid(g1[:, HID:2 * HID])
    gg1 = jnp.tanh(g1[:, 2 * HID:3 * HID])
    oo1 = jax.nn.sigmoid(g1[:, 3 * HID:4 * HID])
    cn1 = ff1 * c1[...] + ii1 * gg1
    c1[...] = cn1
    hn1 = oo1 * jnp.tanh(cn1)
    h1[...] = hn1

    @pl.when(i == LV - 1)
    def _():
        out_ref[...] = hn1


def _tc_lstm(visited, z_all, wi0, wh0, bi0, bh0, wi1, wh1, bi1, bh1):
    wspec = pl.BlockSpec((HID, 4 * HID), lambda i, vis: (0, 0))
    bspec = pl.BlockSpec((1, 4 * HID), lambda i, vis: (0, 0))
    grid_spec = pltpu.PrefetchScalarGridSpec(
        num_scalar_prefetch=1,
        grid=(LV,),
        in_specs=[
            pl.BlockSpec((1, 1, HID), lambda i, vis: (vis[i], 0, 0)),
            wspec, wspec, bspec, bspec, wspec, wspec, bspec, bspec,
        ],
        out_specs=pl.BlockSpec((1, HID), lambda i, vis: (0, 0)),
        scratch_shapes=[pltpu.VMEM((1, HID), F32)] * 4,
    )
    return pl.pallas_call(
        _lstm_body,
        grid_spec=grid_spec,
        out_shape=jax.ShapeDtypeStruct((1, HID), F32),
    )(visited, z_all, wi0, wh0, bi0, bh0, wi1, wh1, bi1, bh1)


def _score_body(vis_ref, z3_ref, fh_ref, g_ref, probs_ref, idx_ref):
    z = z3_ref[...]
    fh = fh_ref[...]
    s = jnp.sum(z * fh[0][None, None, :], axis=2)
    row = lax.broadcasted_iota(I32, (NP // 128, 128), 0)
    col = lax.broadcasted_iota(I32, (NP // 128, 128), 1)
    gidx = row * 128 + col

    s = jnp.where(gidx >= N, -jnp.inf, s)

    def mk(j, sv):
        return jnp.where(gidx == vis_ref[j], -jnp.inf, sv)
    s = lax.fori_loop(0, LV, mk, s)

    mx = jnp.max(s)
    ex = jnp.exp(s - mx)
    probs_ref[...] = ex / jnp.sum(ex)

    v2 = s + g_ref[...]
    mx2 = jnp.max(v2)
    flat = jnp.where(v2 == mx2, gidx, jnp.int32(2 ** 30))
    idx_ref[...] = jnp.full((1, 1), jnp.min(flat), I32)


def _tc_scores(visited, z3, fh, g2d):
    grid_spec = pltpu.PrefetchScalarGridSpec(
        num_scalar_prefetch=1,
        grid=(1,),
        in_specs=[
            pl.BlockSpec((NP // 128, 128, 128), lambda i, vis: (0, 0, 0)),
            pl.BlockSpec((1, HID), lambda i, vis: (0, 0)),
            pl.BlockSpec((NP // 128, 128), lambda i, vis: (0, 0)),
        ],
        out_specs=[
            pl.BlockSpec((NP // 128, 128), lambda i, vis: (0, 0)),
            pl.BlockSpec((1, 1), lambda i, vis: (0, 0)),
        ],
    )
    return pl.pallas_call(
        _score_body,
        grid_spec=grid_spec,
        out_shape=[
            jax.ShapeDtypeStruct((NP // 128, 128), F32),
            jax.ShapeDtypeStruct((1, 1), I32),
        ],
    )(visited, z3, fh, g2d)


def kernel(x, edge_index, edge_weight, visited,
           W1, b1, W_ih0, W_hh0, b_ih0, b_hh0, W_ih1, W_hh1, b_ih1, b_hh1):
    src = edge_index[0]
    dst = edge_index[1]
    pad = EP - E
    pidx = (jnp.arange(pad, dtype=I32) * 37) % N
    srcp = jnp.concatenate([src, pidx])
    dstp = jnp.concatenate([dst, pidx])
    ewp = jnp.concatenate([edge_weight, jnp.zeros((pad,), F32)])

    src32 = srcp.reshape(NW, EPW_C, EC)
    dst32 = dstp.reshape(NW, EPW_C, EC)
    ew32 = ewp.reshape(NW, EPW_C, EC)
    dst16 = dstp.reshape(NS, EPW16_C, EC)
    ew16 = ewp.reshape(NS, EPW16_C, EC)

    x_p = jnp.concatenate([x, jnp.zeros((NP - N, D), F32)])

    deg = _sc_deg(dst16, ew16)
    dinv = _tc_rsqrt(deg.reshape(NP // 128, 128)).reshape(NP)
    h = _tc_matmul(x_p, W1)
    parts = _sc_aggregate(src32, dst32, ew32, dinv, h)
    z_all = _tc_zall(parts[0], parts[1], dinv.reshape(NP, 1), h,
                     b1.reshape(1, HID))

    fh = _tc_lstm(visited, z_all.reshape(NP, 1, HID),
                  W_ih0.T, W_hh0.T, b_ih0.reshape(1, 4 * HID),
                  b_hh0.reshape(1, 4 * HID),
                  W_ih1.T, W_hh1.T, b_ih1.reshape(1, 4 * HID),
                  b_hh1.reshape(1, 4 * HID))

    g = jax.random.gumbel(jax.random.key(1), (N,), F32)
    g2d = jnp.concatenate([g, jnp.zeros((NP - N,), F32)]).reshape(NP // 128, 128)

    probs2d, idx = _tc_scores(visited, z_all.reshape(NP // 128, 128, 128),
                              fh, g2d)
    probs = probs2d.reshape(NP)[:N]
    next_node = idx[0, 0]
    return (next_node, probs)


# speedup vs baseline: 1.8892x; 1.1500x over previous
import jax
import jax.numpy as jnp
from jax import lax
from jax.experimental import pallas as pl
from jax.experimental.pallas import tpu as pltpu
from jax.experimental.pallas import tpu_sc as plsc

N = 10000
E = 320000
D = 128
HID = 128
LV = 100

NP = 10240
NC = 2
NS = 16
NW = NC * NS
EC = 128
EPW_C = 80
EPW = EPW_C * EC
EP = NW * EPW
EPW16_C = 2 * EPW_C
RPT = NP // NS
F32 = jnp.float32
I32 = jnp.int32

_mesh = lambda: plsc.VectorSubcoreMesh(
    core_axis_name="c", subcore_axis_name="s", num_cores=NC, num_subcores=NS)


def _deg_body(dst_hbm, ew_hbm, dinv_hbm, dstb, ewb, vecb, deg_sh):
    c = lax.axis_index("c")
    t = lax.axis_index("s")

    @pl.when(c == 0)
    def _():
        def fill(j, carry):
            vecb[pl.ds(j * 16, 16)] = jnp.full((16,), 1.0, F32)
            return carry
        lax.fori_loop(0, RPT // 16, fill, None)
        pltpu.sync_copy(vecb, deg_sh.at[pl.ds(t * RPT, RPT)])

    plsc.subcore_barrier()

    @pl.when(c == 0)
    def _():
        pltpu.sync_copy(dst_hbm.at[t], dstb)
        pltpu.sync_copy(ew_hbm.at[t], ewb)

        def body(j, carry):
            pltpu.sync_copy(ewb.at[j], deg_sh.at[dstb.at[j]], add=True)
            return carry
        lax.fori_loop(0, EPW16_C, body, None)

    plsc.subcore_barrier()

    @pl.when(c == 0)
    def _():
        pltpu.sync_copy(deg_sh.at[pl.ds(t * RPT, RPT)],
                        dinv_hbm.at[pl.ds(t * RPT, RPT)])


def _sc_deg(dst16, ew16):
    return pl.kernel(
        _deg_body,
        out_type=jax.ShapeDtypeStruct((NP,), F32),
        mesh=_mesh(),
        compiler_params=pltpu.CompilerParams(needs_layout_passes=False),
        scratch_types=[
            pltpu.VMEM((EPW16_C, EC), I32),
            pltpu.VMEM((EPW16_C, EC), F32),
            pltpu.VMEM((RPT,), F32),
            pltpu.VMEM_SHARED((NP,), F32),
        ],
    )(dst16, ew16)


def _agg_body(src_hbm, dst_hbm, ew_hbm, dinv_hbm, h_hbm, out_hbm,
              esrc, edst, eew, normc, rows, dinvv,
              seme, semg, acc):
    c = lax.axis_index("c")
    t = lax.axis_index("s")
    wid = t * NC + c

    pltpu.sync_copy(dinv_hbm, dinvv)

    def zrow(r, carry):
        for cc in range(8):
            rows[0, r, pl.ds(cc * 16, 16)] = jnp.zeros((16,), F32)
        return carry
    lax.fori_loop(0, EC, zrow, None)
    for k in range(RPT // EC):
        pltpu.sync_copy(rows.at[0], acc.at[pl.ds(t * RPT + k * EC, EC)])

    plsc.subcore_barrier()

    def edge_load(j, b):
        pltpu.async_copy(src_hbm.at[wid, j], esrc.at[b], seme)
        pltpu.async_copy(dst_hbm.at[wid, j], edst.at[b], seme)
        pltpu.async_copy(ew_hbm.at[wid, j], eew.at[b], seme)

    def edge_wait(b):
        pltpu.make_async_copy(src_hbm.at[wid, 0], esrc.at[b], seme).wait()
        pltpu.make_async_copy(dst_hbm.at[wid, 0], edst.at[b], seme).wait()
        pltpu.make_async_copy(ew_hbm.at[wid, 0], eew.at[b], seme).wait()

    edge_load(jnp.int32(0), jnp.int32(0))
    edge_wait(jnp.int32(0))
    pltpu.async_copy(h_hbm.at[esrc.at[jnp.int32(0)]], rows.at[jnp.int32(0)],
                     semg)

    def chunk(j, carry):
        b = j & 1
        nb = 1 - b
        for v in range(8):
            sl = pl.ds(v * 16, 16)
            normc[sl] = (eew[b, sl]
                         * plsc.load_gather(dinvv, [esrc[b, sl]])
                         * plsc.load_gather(dinvv, [edst[b, sl]]))

        @pl.when(j + 1 < EPW_C)
        def _():
            edge_load(j + 1, nb)

        pltpu.make_async_copy(h_hbm.at[esrc.at[b]], rows.at[b], semg).wait()
        for e in range(EC):
            sp = plsc.load_gather(normc, [jnp.full((16,), e, I32)])
            for cc in range(8):
                csl = pl.ds(cc * 16, 16)
                rows[b, e, csl] = rows[b, e, csl] * sp

        @pl.when(j + 1 < EPW_C)
        def _():
            edge_wait(nb)
            pltpu.async_copy(h_hbm.at[esrc.at[nb]], rows.at[nb], semg)

        pltpu.sync_copy(rows.at[b], acc.at[edst.at[b]], add=True)
        return carry
    lax.fori_loop(0, EPW_C, chunk, None)

    plsc.subcore_barrier()

    for k in range(RPT // EC):
        sl = pl.ds(t * RPT + k * EC, EC)
        pltpu.sync_copy(acc.at[sl], out_hbm.at[c, sl])


def _sc_aggregate(src32, dst32, ew32, dinv, h):
    return pl.kernel(
        _agg_body,
        out_type=jax.ShapeDtypeStruct((NC, NP, HID), F32),
        mesh=_mesh(),
        compiler_params=pltpu.CompilerParams(needs_layout_passes=False),
        scratch_types=[
            pltpu.VMEM((2, EC), I32),
            pltpu.VMEM((2, EC), I32),
            pltpu.VMEM((2, EC), F32),
            pltpu.VMEM((EC,), F32),
            pltpu.VMEM((2, EC, HID), F32),
            pltpu.VMEM((NP,), F32),
            pltpu.SemaphoreType.DMA, pltpu.SemaphoreType.DMA,
            pltpu.VMEM_SHARED((NP, HID), F32),
        ],
    )(src32, dst32, ew32, dinv, h)


def _rsqrt_body(d_ref, o_ref):
    o_ref[...] = lax.rsqrt(d_ref[...])


def _tc_rsqrt(deg2d):
    return pl.pallas_call(
        _rsqrt_body,
        out_shape=jax.ShapeDtypeStruct((NP // 128, 128), F32),
    )(deg2d)


def _mm_body(x_ref, w_ref, o_ref):
    o_ref[...] = jnp.dot(x_ref[...], w_ref[...], preferred_element_type=F32)


def _tc_matmul(x_p, W1):
    return pl.pallas_call(
        _mm_body,
        grid=(NP // 512,),
        in_specs=[
            pl.BlockSpec((512, D), lambda i: (i, 0)),
            pl.BlockSpec((D, HID), lambda i: (0, 0)),
        ],
        out_specs=pl.BlockSpec((512, HID), lambda i: (i, 0)),
        out_shape=jax.ShapeDtypeStruct((NP, HID), F32),
    )(x_p, W1)


def _zall_body(p0_ref, p1_ref, dv_ref, h_ref, b_ref, o_ref):
    dv = dv_ref[...]
    o_ref[...] = p0_ref[...] + p1_ref[...] + (dv * dv) * h_ref[...] + b_ref[...]


def _tc_zall(p0, p1, dinv_col, h, b1_row):
    blk = 512
    return pl.pallas_call(
        _zall_body,
        grid=(NP // blk,),
        in_specs=[
            pl.BlockSpec((blk, HID), lambda i: (i, 0)),
            pl.BlockSpec((blk, HID), lambda i: (i, 0)),
            pl.BlockSpec((blk, 1), lambda i: (i, 0)),
            pl.BlockSpec((blk, HID), lambda i: (i, 0)),
            pl.BlockSpec((1, HID), lambda i: (0, 0)),
        ],
        out_specs=pl.BlockSpec((blk, HID), lambda i: (i, 0)),
        out_shape=jax.ShapeDtypeStruct((NP, HID), F32),
    )(p0, p1, dinv_col, h, b1_row)


def _lstm_body(vis_ref, z_ref, wi0, wh0, bi0, bh0, wi1, wh1, bi1, bh1,
               out_ref, h0, c0, h1, c1):
    i = pl.program_id(0)

    @pl.when(i == 0)
    def _():
        h0[...] = jnp.zeros((1, HID), F32)
        c0[...] = jnp.zeros((1, HID), F32)
        h1[...] = jnp.zeros((1, HID), F32)
        c1[...] = jnp.zeros((1, HID), F32)

    x = z_ref[0]
    g = (jnp.dot(x, wi0[...], preferred_element_type=F32)
         + jnp.dot(h0[...], wh0[...], preferred_element_type=F32)
         + bi0[...] + bh0[...])
    ii = jax.nn.sigmoid(g[:, 0:HID])
    ff = jax.nn.sigmoid(g[:, HID:2 * HID])
    gg = jnp.tanh(g[:, 2 * HID:3 * HID])
    oo = jax.nn.sigmoid(g[:, 3 * HID:4 * HID])
    cn0 = ff * c0[...] + ii * gg
    c0[...] = cn0
    hn0 = oo * jnp.tanh(cn0)
    h0[...] = hn0

    g1 = (jnp.dot(hn0, wi1[...], preferred_element_type=F32)
          + jnp.dot(h1[...], wh1[...], preferred_element_type=F32)
          + bi1[...] + bh1[...])
    ii1 = jax.nn.sigmoid(g1[:, 0:HID])
    ff1 = jax.nn.sigmoid(g1[:, HID:2 * HID])
    gg1 = jnp.tanh(g1[:, 2 * HID:3 * HID])
    oo1 = jax.nn.sigmoid(g1[:, 3 * HID:4 * HID])
    cn1 = ff1 * c1[...] + ii1 * gg1
    c1[...] = cn1
    hn1 = oo1 * jnp.tanh(cn1)
    h1[...] = hn1

    @pl.when(i == LV - 1)
    def _():
        out_ref[...] = hn1


def _tc_lstm(visited, z_all, wi0, wh0, bi0, bh0, wi1, wh1, bi1, bh1):
    wspec = pl.BlockSpec((HID, 4 * HID), lambda i, vis: (0, 0))
    bspec = pl.BlockSpec((1, 4 * HID), lambda i, vis: (0, 0))
    grid_spec = pltpu.PrefetchScalarGridSpec(
        num_scalar_prefetch=1,
        grid=(LV,),
        in_specs=[
            pl.BlockSpec((1, 1, HID), lambda i, vis: (vis[i], 0, 0)),
            wspec, wspec, bspec, bspec, wspec, wspec, bspec, bspec,
        ],
        out_specs=pl.BlockSpec((1, HID), lambda i, vis: (0, 0)),
        scratch_shapes=[pltpu.VMEM((1, HID), F32)] * 4,
    )
    return pl.pallas_call(
        _lstm_body,
        grid_spec=grid_spec,
        out_shape=jax.ShapeDtypeStruct((1, HID), F32),
    )(visited, z_all, wi0, wh0, bi0, bh0, wi1, wh1, bi1, bh1)


def _score_body(vis_ref, z3_ref, fh_ref, g_ref, probs_ref, idx_ref):
    z = z3_ref[...]
    fh = fh_ref[...]
    s = jnp.sum(z * fh[0][None, None, :], axis=2)
    row = lax.broadcasted_iota(I32, (NP // 128, 128), 0)
    col = lax.broadcasted_iota(I32, (NP // 128, 128), 1)
    gidx = row * 128 + col

    s = jnp.where(gidx >= N, -jnp.inf, s)

    def mk(j, sv):
        return jnp.where(gidx == vis_ref[j], -jnp.inf, sv)
    s = lax.fori_loop(0, LV, mk, s)

    mx = jnp.max(s)
    ex = jnp.exp(s - mx)
    probs_ref[...] = ex / jnp.sum(ex)

    v2 = s + g_ref[...]
    mx2 = jnp.max(v2)
    flat = jnp.where(v2 == mx2, gidx, jnp.int32(2 ** 30))
    idx_ref[...] = jnp.full((1, 1), jnp.min(flat), I32)


def _tc_scores(visited, z3, fh, g2d):
    grid_spec = pltpu.PrefetchScalarGridSpec(
        num_scalar_prefetch=1,
        grid=(1,),
        in_specs=[
            pl.BlockSpec((NP // 128, 128, 128), lambda i, vis: (0, 0, 0)),
            pl.BlockSpec((1, HID), lambda i, vis: (0, 0)),
            pl.BlockSpec((NP // 128, 128), lambda i, vis: (0, 0)),
        ],
        out_specs=[
            pl.BlockSpec((NP // 128, 128), lambda i, vis: (0, 0)),
            pl.BlockSpec((1, 1), lambda i, vis: (0, 0)),
        ],
    )
    return pl.pallas_call(
        _score_body,
        grid_spec=grid_spec,
        out_shape=[
            jax.ShapeDtypeStruct((NP // 128, 128), F32),
            jax.ShapeDtypeStruct((1, 1), I32),
        ],
    )(visited, z3, fh, g2d)


def kernel(x, edge_index, edge_weight, visited,
           W1, b1, W_ih0, W_hh0, b_ih0, b_hh0, W_ih1, W_hh1, b_ih1, b_hh1):
    src = edge_index[0]
    dst = edge_index[1]
    pad = EP - E
    pidx = (jnp.arange(pad, dtype=I32) * 37) % N
    srcp = jnp.concatenate([src, pidx])
    dstp = jnp.concatenate([dst, pidx])
    ewp = jnp.concatenate([edge_weight, jnp.zeros((pad,), F32)])

    src32 = srcp.reshape(NW, EPW_C, EC)
    dst32 = dstp.reshape(NW, EPW_C, EC)
    ew32 = ewp.reshape(NW, EPW_C, EC)
    dst16 = dstp.reshape(NS, EPW16_C, EC)
    ew16 = ewp.reshape(NS, EPW16_C, EC)

    x_p = jnp.concatenate([x, jnp.zeros((NP - N, D), F32)])

    deg = _sc_deg(dst16, ew16)
    dinv = _tc_rsqrt(deg.reshape(NP // 128, 128)).reshape(NP)
    h = _tc_matmul(x_p, W1)
    parts = _sc_aggregate(src32, dst32, ew32, dinv, h)
    z_all = _tc_zall(parts[0], parts[1], dinv.reshape(NP, 1), h,
                     b1.reshape(1, HID))

    fh = _tc_lstm(visited, z_all.reshape(NP, 1, HID),
                  W_ih0.T, W_hh0.T, b_ih0.reshape(1, 4 * HID),
                  b_hh0.reshape(1, 4 * HID),
                  W_ih1.T, W_hh1.T, b_ih1.reshape(1, 4 * HID),
                  b_hh1.reshape(1, 4 * HID))

    g = jax.random.gumbel(jax.random.key(1), (N,), F32)
    g2d = jnp.concatenate([g, jnp.zeros((NP - N,), F32)]).reshape(NP // 128, 128)

    probs2d, idx = _tc_scores(visited, z_all.reshape(NP // 128, 128, 128),
                              fh, g2d)
    probs = probs2d.reshape(NP)[:N]
    next_node = idx[0, 0]
    return (next_node, probs)
